# Initial kernel scaffold; baseline (speedup 1.0000x reference)
import jax
import jax.numpy as jnp
from jax.experimental import pallas as pl


def kernel(grid_node_feats, params, mesh_node_feats, mesh_edge_feats, g2m_edge_feats, m2g_edge_feats, g2m_src_idx, g2m_dst_idx, m2m_src_idx, m2m_dst_idx, m2g_src_idx, m2g_dst_idx, per_variable_level_mean, per_variable_level_std):
    raise NotImplementedError("write your pallas kernel here")


# speedup vs baseline: 1.1481x; 1.1481x over previous
import functools

import jax
import jax.numpy as jnp
from jax import lax
from jax.experimental import pallas as pl
from jax.experimental.pallas import tpu as pltpu

_LAT = 128


def _fused_mlp(xs, w1s, b1, w2, b2, g=None, be=None, res=None, final=None,
               bn=2000):
    n = xs[0].shape[0]
    dout = w2.shape[1]
    assert n % bn == 0, (n, bn)
    nx = len(xs)
    has_ln = g is not None
    has_res = res is not None
    has_final = final is not None

    ops = list(xs) + list(w1s) + [b1.reshape(1, -1), w2, b2.reshape(1, -1)]
    if has_ln:
        ops += [g.reshape(1, -1), be.reshape(1, -1)]
    if has_res:
        ops += [res]
    if has_final:
        std, mean, xfull = final
        ops += [std.reshape(1, -1), mean.reshape(1, -1), xfull]

    def body(*refs):
        xs_r = refs[:nx]
        w1_r = refs[nx:2 * nx]
        i = 2 * nx
        b1_r, w2_r, b2_r = refs[i], refs[i + 1], refs[i + 2]
        i += 3
        if has_ln:
            g_r, be_r = refs[i], refs[i + 1]
            i += 2
        if has_res:
            res_r = refs[i]
            i += 1
        if has_final:
            std_r, mean_r, xf_r = refs[i], refs[i + 1], refs[i + 2]
            i += 3
        out_r = refs[i]

        acc = None
        for xr, wr in zip(xs_r, w1_r):
            x = xr[...]
            w = wr[...]
            if x.shape[1] < 8:
                part = x[:, 0:1] * w[0:1, :]
                for j in range(1, x.shape[1]):
                    part = part + x[:, j:j + 1] * w[j:j + 1, :]
            else:
                part = jnp.dot(x, w, preferred_element_type=jnp.float32)
            acc = part if acc is None else acc + part
        h = acc + b1_r[...]
        h = h * jax.nn.sigmoid(h)
        y = jnp.dot(h, w2_r[...], preferred_element_type=jnp.float32) + b2_r[...]
        if has_ln:
            mu = jnp.mean(y, axis=-1, keepdims=True)
            var = jnp.mean((y - mu) * (y - mu), axis=-1, keepdims=True)
            y = (y - mu) * lax.rsqrt(var + 1e-5) * g_r[...] + be_r[...]
        if has_res:
            y = y + res_r[...]
        if has_final:
            y = y * std_r[...] + mean_r[...] + xf_r[:, dout:2 * dout]
        out_r[...] = y

    in_specs = []
    for x in xs:
        in_specs.append(pl.BlockSpec((bn, x.shape[1]), lambda i: (i, 0)))
    for w in w1s:
        in_specs.append(pl.BlockSpec(w.shape, lambda i: (0, 0)))
    in_specs.append(pl.BlockSpec((1, b1.shape[-1]), lambda i: (0, 0)))
    in_specs.append(pl.BlockSpec(w2.shape, lambda i: (0, 0)))
    in_specs.append(pl.BlockSpec((1, dout), lambda i: (0, 0)))
    if has_ln:
        in_specs.append(pl.BlockSpec((1, dout), lambda i: (0, 0)))
        in_specs.append(pl.BlockSpec((1, dout), lambda i: (0, 0)))
    if has_res:
        in_specs.append(pl.BlockSpec((bn, dout), lambda i: (i, 0)))
    if has_final:
        in_specs.append(pl.BlockSpec((1, dout), lambda i: (0, 0)))
        in_specs.append(pl.BlockSpec((1, dout), lambda i: (0, 0)))
        in_specs.append(pl.BlockSpec((bn, xfull.shape[1]), lambda i: (i, 0)))

    return pl.pallas_call(
        body,
        grid=(n // bn,),
        in_specs=in_specs,
        out_specs=pl.BlockSpec((bn, dout), lambda i: (i, 0)),
        out_shape=jax.ShapeDtypeStruct((n, dout), jnp.float32),
    )(*ops)


def _mlp_p(p, xs, res=None, final=None, bn=2000):
    dins = [x.shape[1] for x in xs]
    w1 = p['w1']
    w1s = []
    o = 0
    for d in dins:
        w1s.append(w1[o:o + d])
        o += d
    g = p.get('g')
    be = p.get('be')
    return _fused_mlp(xs, w1s, p['b1'], p['w2'], p['b2'], g=g, be=be, res=res,
                      final=final, bn=bn)


def kernel(grid_node_feats, params, mesh_node_feats, mesh_edge_feats,
           g2m_edge_feats, m2g_edge_feats, g2m_src_idx, g2m_dst_idx,
           m2m_src_idx, m2m_dst_idx, m2g_src_idx, m2g_dst_idx,
           per_variable_level_mean, per_variable_level_std):
    x = grid_node_feats[0]
    n_grid = x.shape[0]
    n_mesh = mesh_node_feats.shape[0]

    vg = _mlp_p(params['enc_vg'], [x])
    vm = _mlp_p(params['enc_vm'], [mesh_node_feats])
    em = _mlp_p(params['enc_em'], [mesh_edge_feats])
    eg2m = _mlp_p(params['enc_eg2m'], [g2m_edge_feats])
    em2g = _mlp_p(params['enc_em2g'], [m2g_edge_feats])

    gsrc = vg[g2m_src_idx]
    gdst = vm[g2m_dst_idx]
    eg2m = _mlp_p(params['g2m_edge'], [eg2m, gsrc, gdst], res=eg2m)
    agg = jax.ops.segment_sum(eg2m, g2m_dst_idx, num_segments=n_mesh)
    vm = _mlp_p(params['g2m_node'], [vm, agg], res=vm)
    vg = _mlp_p(params['g2m_grid'], [vg], res=vg)

    for sp in params['proc']:
        gsrc = vm[m2m_src_idx]
        gdst = vm[m2m_dst_idx]
        em = _mlp_p(sp['edge'], [em, gsrc, gdst], res=em)
        agg = jax.ops.segment_sum(em, m2m_dst_idx, num_segments=n_mesh)
        vm = _mlp_p(sp['node'], [vm, agg], res=vm)

    gsrc = vm[m2g_src_idx]
    gdst = vg[m2g_dst_idx]
    em2g = _mlp_p(params['dec_edge'], [em2g, gsrc, gdst], res=em2g)
    agg = jax.ops.segment_sum(em2g, m2g_dst_idx, num_segments=n_grid)
    vg = _mlp_p(params['dec_node'], [vg, agg], res=vg)
    out = _mlp_p(params['dec_final'], [vg],
                 final=(per_variable_level_std, per_variable_level_mean, x))
    return out[None]
